# baseline (speedup 1.0000x reference)
import functools

import jax
import jax.numpy as jnp
from jax import lax
from jax.experimental import pallas as pl
from jax.experimental.pallas import tpu as pltpu
from jax.experimental.pallas import tpu_sc as plsc

N = 10000
C = 256
E = 160000

NC = 2
NS = 16
LANES = 16

NP = 10240
EP = 165888
ED = EP // NS
NG = ED // 128
CH = C // 2
HALF = N // NC
ACCR = 5120

_MESH = plsc.VectorSubcoreMesh(core_axis_name="c", subcore_axis_name="s",
                               num_cores=NC, num_subcores=NS)
_SC_PARAMS = pltpu.CompilerParams(needs_layout_passes=False)


def _newton_rsqrt(x):
    i = plsc.bitcast(x, jnp.int32)
    y = plsc.bitcast(jnp.int32(0x5F3759DF) - (i >> 1), jnp.float32)
    for _ in range(3):
        y = y * (1.5 - 0.5 * x * y * y)
    return y


@functools.partial(
    pl.kernel,
    out_type=jax.ShapeDtypeStruct((NP,), jnp.float32),
    mesh=_MESH,
    compiler_params=_SC_PARAMS,
    scratch_types=[
        pltpu.VMEM((ED,), jnp.int32),
        pltpu.VMEM((ED,), jnp.float32),
        pltpu.VMEM((NP,), jnp.float32),
        pltpu.VMEM((NP,), jnp.float32),
        pltpu.VMEM((NP // NS,), jnp.float32),
        pltpu.VMEM_SHARED((NS * NP,), jnp.float32),
    ],
)
def _deg_dinv(dst_hbm, ew_hbm, dinv_hbm, dstb, ewb, degl, sbuf, dinvv, spart):
    cid = lax.axis_index("c")
    sid = lax.axis_index("s")
    pltpu.sync_copy(dst_hbm.at[pl.ds(sid * ED, ED)], dstb)
    pltpu.sync_copy(ew_hbm.at[pl.ds(sid * ED, ED)], ewb)

    def zero_body(i, _):
        degl[pl.ds(i * LANES, LANES)] = jnp.zeros((LANES,), jnp.float32)
        return 0
    lax.fori_loop(0, NP // LANES, zero_body, 0)

    def edge_body(i, _):
        sl = pl.ds(i * LANES, LANES)
        plsc.addupdate_scatter(degl, [dstb[sl]], ewb[sl])
        return 0
    lax.fori_loop(0, ED // LANES, edge_body, 0)

    pltpu.sync_copy(degl, spart.at[pl.ds(sid * NP, NP)])
    plsc.subcore_barrier()

    nsl = NP // NS
    base = sid * nsl
    for t in range(NS):
        pltpu.sync_copy(spart.at[pl.ds(t * NP + base, nsl)],
                        sbuf.at[pl.ds(t * nsl, nsl)])

    def red_body(k, _):
        acc = jnp.full((LANES,), 1.0, jnp.float32)
        for t in range(NS):
            acc = acc + sbuf[pl.ds(t * nsl + k * LANES, LANES)]
        dinvv[pl.ds(k * LANES, LANES)] = _newton_rsqrt(acc)
        return 0
    lax.fori_loop(0, nsl // LANES, red_body, 0)

    @pl.when(cid == 0)
    def _():
        pltpu.sync_copy(dinvv, dinv_hbm.at[pl.ds(base, nsl)])


@functools.partial(
    pl.kernel,
    out_type=jax.ShapeDtypeStruct((N, C), jnp.float32),
    mesh=_MESH,
    compiler_params=_SC_PARAMS,
    scratch_types=[
        pltpu.VMEM((ED,), jnp.int32),
        pltpu.VMEM((ED,), jnp.int32),
        pltpu.VMEM((ED,), jnp.float32),
        pltpu.VMEM((NP,), jnp.float32),
        pltpu.VMEM((C,), jnp.float32),
        pltpu.VMEM((ED,), jnp.float32),
        pltpu.VMEM((NG, 128), jnp.int32),
        pltpu.VMEM((128, CH), jnp.float32),
        pltpu.VMEM((8, CH), jnp.float32),
        pltpu.VMEM((8, CH), jnp.float32),
        pltpu.VMEM_SHARED((ACCR, CH), jnp.float32),
    ],
)
def _gcn_agg(src_hbm, dst_hbm, ew_hbm, dinv_hbm, xh0_hbm, xh1_hbm, bg_hbm,
             out_hbm, srcb, dstc, ewc, dinvb, bgb, scaleb, idxb,
             rows0, dchunk, xchunk, acc):
    cid = lax.axis_index("c")
    sid = lax.axis_index("s")
    base = cid * HALF

    pltpu.sync_copy(src_hbm.at[pl.ds(sid * ED, ED)], srcb)
    pltpu.sync_copy(dst_hbm.at[pl.ds(sid * ED, ED)], dstc)
    pltpu.sync_copy(ew_hbm.at[pl.ds(sid * ED, ED)], ewc)
    pltpu.sync_copy(dinv_hbm, dinvb)
    pltpu.sync_copy(bg_hbm, bgb)

    def prep(g, _):
        off = g * 128
        for k in range(8):
            sl = pl.ds(off + k * LANES, LANES)
            dl = dstc[sl] - base
            ok = (dl >= 0) & (dl < HALF)
            sv = plsc.load_gather(dinvb, [srcb[sl]])
            scaleb[sl] = sv * ewc[sl]
            idxb[g, pl.ds(k * LANES, LANES)] = jnp.where(ok, dl, HALF + sid)
        return 0
    lax.fori_loop(0, NG, prep, 0)

    def scale_rows(g, rb):
        def scale_row(e, _):
            sv = plsc.load_gather(
                scaleb, [jnp.full((LANES,), g * 128 + e, jnp.int32)])
            for r in range(CH // LANES):
                sl2 = pl.ds(r * LANES, LANES)
                rb[e, sl2] = rb[e, sl2] * sv
            return 0
        lax.fori_loop(0, 128, scale_row, 0)

    zr0 = sid * (ACCR // NS)
    for p in range(2):
        xw_hbm = xh0_hbm if p == 0 else xh1_hbm

        def zrow(r, _):
            for k in range(CH // LANES):
                rows0[r, pl.ds(k * LANES, LANES)] = jnp.zeros((LANES,),
                                                              jnp.float32)
            return 0
        lax.fori_loop(0, 128, zrow, 0)
        pltpu.sync_copy(rows0, acc.at[pl.ds(zr0, 128)])
        pltpu.sync_copy(rows0, acc.at[pl.ds(zr0 + 128, 128)])
        pltpu.sync_copy(rows0.at[pl.ds(0, 64)], acc.at[pl.ds(zr0 + 256, 64)])
        plsc.subcore_barrier()

        def group(g, _):
            pltpu.sync_copy(xw_hbm.at[srcb.at[pl.ds(g * 128, 128)]], rows0)
            scale_rows(g, rows0)
            pltpu.sync_copy(rows0, acc.at[idxb.at[g]], add=True)
            return 0
        lax.fori_loop(0, NG, group, 0)
        plsc.subcore_barrier()

        def drain(k, _):
            row0 = zr0 + k * 8

            @pl.when(row0 < HALF)
            def _():
                g0 = base + row0
                pltpu.sync_copy(acc.at[pl.ds(row0, 8)], dchunk)
                pltpu.sync_copy(xw_hbm.at[pl.ds(g0, 8)], xchunk)
                for rr in range(8):
                    dval = plsc.load_gather(
                        dinvb, [jnp.full((LANES,), g0 + rr, jnp.int32)])
                    for r in range(CH // LANES):
                        sl2 = pl.ds(r * LANES, LANES)
                        dchunk[rr, sl2] = (dval * (dchunk[rr, sl2]
                                                   + dval * xchunk[rr, sl2])
                                           + bgb[pl.ds(p * CH + r * LANES,
                                                       LANES)])
                pltpu.sync_copy(dchunk,
                                out_hbm.at[pl.ds(g0, 8), pl.ds(p * CH, CH)])
            return 0
        lax.fori_loop(0, (ACCR // NS) // 8, drain, 0)
        plsc.subcore_barrier()


_BLK = 1000


def _glu_mm_body(x_ref, w1_ref, b1_ref, wg_ref, o0_ref, o1_ref):
    h = jnp.dot(x_ref[...], w1_ref[...],
                preferred_element_type=jnp.float32) + b1_ref[...][None, :]
    act = h[:, :C] * jax.nn.sigmoid(h[:, C:])
    xw = jnp.dot(act, wg_ref[...], preferred_element_type=jnp.float32)
    o0_ref[...] = xw[:, :CH]
    o1_ref[...] = xw[:, CH:]


def _glu2_body(t_ref, w2_ref, b2_ref, o_ref):
    h = jnp.dot(t_ref[...], w2_ref[...],
                preferred_element_type=jnp.float32) + b2_ref[...][None, :]
    o_ref[...] = h[:, :C] * jax.nn.sigmoid(h[:, C:])


def _glu_mm(x2, W1, b1, Wg):
    return pl.pallas_call(
        _glu_mm_body,
        grid=(N // _BLK,),
        in_specs=[
            pl.BlockSpec((_BLK, C), lambda i: (i, 0)),
            pl.BlockSpec((C, 2 * C), lambda i: (0, 0)),
            pl.BlockSpec((2 * C,), lambda i: (0,)),
            pl.BlockSpec((C, C), lambda i: (0, 0)),
        ],
        out_specs=[pl.BlockSpec((_BLK, CH), lambda i: (i, 0)),
                   pl.BlockSpec((_BLK, CH), lambda i: (i, 0))],
        out_shape=[jax.ShapeDtypeStruct((N, CH), jnp.float32),
                   jax.ShapeDtypeStruct((N, CH), jnp.float32)],
    )(x2, W1, b1, Wg)


def _glu2(t, W2, b2):
    return pl.pallas_call(
        _glu2_body,
        grid=(N // _BLK,),
        in_specs=[
            pl.BlockSpec((_BLK, C), lambda i: (i, 0)),
            pl.BlockSpec((C, 2 * C), lambda i: (0, 0)),
            pl.BlockSpec((2 * C,), lambda i: (0,)),
        ],
        out_specs=pl.BlockSpec((_BLK, C), lambda i: (i, 0)),
        out_shape=jax.ShapeDtypeStruct((N, C), jnp.float32),
    )(t, W2, b2)


def kernel(x, edge_index, edge_attr, batch, W1, b1, Wg, bg, W2, b2):
    x2 = x.reshape(N, C)
    pad = EP - E
    srcp = jnp.concatenate([edge_index[0], jnp.zeros((pad,), jnp.int32)])
    dstp = jnp.concatenate([edge_index[1], jnp.zeros((pad,), jnp.int32)])
    ewp = jnp.concatenate([edge_attr, jnp.zeros((pad,), jnp.float32)])

    dinv = _deg_dinv(dstp, ewp)
    xh0, xh1 = _glu_mm(x2, W1, b1, Wg)
    t = _gcn_agg(srcp, dstp, ewp, dinv, xh0, xh1, bg)
    y = _glu2(t, W2, b2)
    return y.reshape(1, 1, N, C)


# speedup vs baseline: 1.5820x; 1.5820x over previous
import functools

import jax
import jax.numpy as jnp
from jax import lax
from jax.experimental import pallas as pl
from jax.experimental.pallas import tpu as pltpu
from jax.experimental.pallas import tpu_sc as plsc

N = 10000
C = 256
E = 160000

NC = 2
NS = 16
LANES = 16

NP = 10240
EP = 161792
ED = EP // NS
NG = ED // 128
CH = C // 2
HALF = N // NC
ACCR = 5120

_MESH = plsc.VectorSubcoreMesh(core_axis_name="c", subcore_axis_name="s",
                               num_cores=NC, num_subcores=NS)
_SC_PARAMS = pltpu.CompilerParams(needs_layout_passes=False)


def _newton_rsqrt(x):
    i = plsc.bitcast(x, jnp.int32)
    y = plsc.bitcast(jnp.int32(0x5F3759DF) - (i >> 1), jnp.float32)
    for _ in range(3):
        y = y * (1.5 - 0.5 * x * y * y)
    return y


@functools.partial(
    pl.kernel,
    out_type=jax.ShapeDtypeStruct((NP,), jnp.float32),
    mesh=_MESH,
    compiler_params=_SC_PARAMS,
    scratch_types=[
        pltpu.VMEM((ED,), jnp.int32),
        pltpu.VMEM((ED,), jnp.float32),
        pltpu.VMEM((NP,), jnp.float32),
        pltpu.VMEM((NP,), jnp.float32),
        pltpu.VMEM((NP // NS,), jnp.float32),
        pltpu.VMEM_SHARED((NS * NP,), jnp.float32),
    ],
)
def _deg_dinv(dst_hbm, ew_hbm, dinv_hbm, dstb, ewb, degl, sbuf, dinvv, spart):
    cid = lax.axis_index("c")
    sid = lax.axis_index("s")
    pltpu.sync_copy(dst_hbm.at[pl.ds(sid * ED, ED)], dstb)
    pltpu.sync_copy(ew_hbm.at[pl.ds(sid * ED, ED)], ewb)

    def zero_body(i, _):
        degl[pl.ds(i * LANES, LANES)] = jnp.zeros((LANES,), jnp.float32)
        return 0
    lax.fori_loop(0, NP // LANES, zero_body, 0)

    def edge_body(i, _):
        sl = pl.ds(i * LANES, LANES)
        plsc.addupdate_scatter(degl, [dstb[sl]], ewb[sl])
        return 0
    lax.fori_loop(0, ED // LANES, edge_body, 0)

    pltpu.sync_copy(degl, spart.at[pl.ds(sid * NP, NP)])
    plsc.subcore_barrier()

    nsl = NP // NS
    base = sid * nsl
    for t in range(NS):
        pltpu.sync_copy(spart.at[pl.ds(t * NP + base, nsl)],
                        sbuf.at[pl.ds(t * nsl, nsl)])

    def red_body(k, _):
        acc = jnp.full((LANES,), 1.0, jnp.float32)
        for t in range(NS):
            acc = acc + sbuf[pl.ds(t * nsl + k * LANES, LANES)]
        dinvv[pl.ds(k * LANES, LANES)] = _newton_rsqrt(acc)
        return 0
    lax.fori_loop(0, nsl // LANES, red_body, 0)

    @pl.when(cid == 0)
    def _():
        pltpu.sync_copy(dinvv, dinv_hbm.at[pl.ds(base, nsl)])


@functools.partial(
    pl.kernel,
    out_type=jax.ShapeDtypeStruct((N, C), jnp.float32),
    mesh=_MESH,
    compiler_params=_SC_PARAMS,
    scratch_types=[
        pltpu.VMEM((ED,), jnp.int32),
        pltpu.VMEM((ED,), jnp.int32),
        pltpu.VMEM((ED,), jnp.float32),
        pltpu.VMEM((NP,), jnp.float32),
        pltpu.VMEM((C,), jnp.float32),
        pltpu.VMEM((ED,), jnp.float32),
        pltpu.VMEM((NG, 128), jnp.int32),
        pltpu.VMEM((128, CH), jnp.float32),
        pltpu.VMEM((8, CH), jnp.float32),
        pltpu.VMEM((8, CH), jnp.float32),
        pltpu.VMEM_SHARED((ACCR, CH), jnp.float32),
    ],
)
def _gcn_agg(src_hbm, dst_hbm, ew_hbm, dinv_hbm, xh0_hbm, xh1_hbm, bg_hbm,
             out_hbm, srcb, dstc, ewc, dinvb, bgb, scaleb, idxb,
             rows0, dchunk, xchunk, acc):
    cid = lax.axis_index("c")
    sid = lax.axis_index("s")
    base = cid * HALF

    pltpu.sync_copy(src_hbm.at[pl.ds(sid * ED, ED)], srcb)
    pltpu.sync_copy(dst_hbm.at[pl.ds(sid * ED, ED)], dstc)
    pltpu.sync_copy(ew_hbm.at[pl.ds(sid * ED, ED)], ewc)
    pltpu.sync_copy(dinv_hbm, dinvb)
    pltpu.sync_copy(bg_hbm, bgb)

    def prep(g, _):
        off = g * 128
        for k in range(8):
            sl = pl.ds(off + k * LANES, LANES)
            dl = dstc[sl] - base
            ok = (dl >= 0) & (dl < HALF)
            sv = plsc.load_gather(dinvb, [srcb[sl]])
            scaleb[sl] = sv * ewc[sl]
            idxb[g, pl.ds(k * LANES, LANES)] = jnp.where(ok, dl, HALF + sid)
        return 0
    lax.fori_loop(0, NG, prep, 0)

    def scale_rows(g, rb):
        def scale_row(e, _):
            sv = plsc.load_gather(
                scaleb, [jnp.full((LANES,), g * 128 + e, jnp.int32)])
            for r in range(CH // LANES):
                sl2 = pl.ds(r * LANES, LANES)
                rb[e, sl2] = rb[e, sl2] * sv
            return 0
        lax.fori_loop(0, 128, scale_row, 0)

    zr0 = sid * (ACCR // NS)
    for p in range(2):
        xw_hbm = xh0_hbm if p == 0 else xh1_hbm

        def zrow(r, _):
            for k in range(CH // LANES):
                rows0[r, pl.ds(k * LANES, LANES)] = jnp.zeros((LANES,),
                                                              jnp.float32)
            return 0
        lax.fori_loop(0, 128, zrow, 0)
        pltpu.sync_copy(rows0, acc.at[pl.ds(zr0, 128)])
        pltpu.sync_copy(rows0, acc.at[pl.ds(zr0 + 128, 128)])
        pltpu.sync_copy(rows0.at[pl.ds(0, 64)], acc.at[pl.ds(zr0 + 256, 64)])
        plsc.subcore_barrier()

        def group(g, _):
            pltpu.sync_copy(xw_hbm.at[srcb.at[pl.ds(g * 128, 128)]], rows0)
            scale_rows(g, rows0)
            pltpu.sync_copy(rows0, acc.at[idxb.at[g]], add=True)
            return 0
        lax.fori_loop(0, NG, group, 0)
        plsc.subcore_barrier()

        nchunks = HALF // 8

        def drain(jj, _):
            j = jj * NS + sid

            @pl.when(j < nchunks)
            def _():
                row0 = j * 8
                g0 = base + row0
                pltpu.sync_copy(acc.at[pl.ds(row0, 8)], dchunk)
                pltpu.sync_copy(xw_hbm.at[pl.ds(g0, 8)], xchunk)
                for rr in range(8):
                    dval = plsc.load_gather(
                        dinvb, [jnp.full((LANES,), g0 + rr, jnp.int32)])
                    for r in range(CH // LANES):
                        sl2 = pl.ds(r * LANES, LANES)
                        dchunk[rr, sl2] = (dval * (dchunk[rr, sl2]
                                                   + dval * xchunk[rr, sl2])
                                           + bgb[pl.ds(p * CH + r * LANES,
                                                       LANES)])
                pltpu.sync_copy(dchunk,
                                out_hbm.at[pl.ds(g0, 8), pl.ds(p * CH, CH)])
            return 0
        lax.fori_loop(0, (nchunks + NS - 1) // NS + 1, drain, 0)
        plsc.subcore_barrier()


_BLK = 1000


def _glu_mm_body(x_ref, w1_ref, b1_ref, wg_ref, o0_ref, o1_ref):
    h = jnp.dot(x_ref[...], w1_ref[...],
                preferred_element_type=jnp.float32) + b1_ref[...][None, :]
    act = h[:, :C] * jax.nn.sigmoid(h[:, C:])
    xw = jnp.dot(act, wg_ref[...], preferred_element_type=jnp.float32)
    o0_ref[...] = xw[:, :CH]
    o1_ref[...] = xw[:, CH:]


def _glu2_body(t_ref, w2_ref, b2_ref, o_ref):
    h = jnp.dot(t_ref[...], w2_ref[...],
                preferred_element_type=jnp.float32) + b2_ref[...][None, :]
    o_ref[...] = h[:, :C] * jax.nn.sigmoid(h[:, C:])


def _glu_mm(x2, W1, b1, Wg):
    return pl.pallas_call(
        _glu_mm_body,
        grid=(N // _BLK,),
        in_specs=[
            pl.BlockSpec((_BLK, C), lambda i: (i, 0)),
            pl.BlockSpec((C, 2 * C), lambda i: (0, 0)),
            pl.BlockSpec((2 * C,), lambda i: (0,)),
            pl.BlockSpec((C, C), lambda i: (0, 0)),
        ],
        out_specs=[pl.BlockSpec((_BLK, CH), lambda i: (i, 0)),
                   pl.BlockSpec((_BLK, CH), lambda i: (i, 0))],
        out_shape=[jax.ShapeDtypeStruct((N, CH), jnp.float32),
                   jax.ShapeDtypeStruct((N, CH), jnp.float32)],
    )(x2, W1, b1, Wg)


def _glu2(t, W2, b2):
    return pl.pallas_call(
        _glu2_body,
        grid=(N // _BLK,),
        in_specs=[
            pl.BlockSpec((_BLK, C), lambda i: (i, 0)),
            pl.BlockSpec((C, 2 * C), lambda i: (0, 0)),
            pl.BlockSpec((2 * C,), lambda i: (0,)),
        ],
        out_specs=pl.BlockSpec((_BLK, C), lambda i: (i, 0)),
        out_shape=jax.ShapeDtypeStruct((N, C), jnp.float32),
    )(t, W2, b2)


def kernel(x, edge_index, edge_attr, batch, W1, b1, Wg, bg, W2, b2):
    x2 = x.reshape(N, C)
    pad = EP - E
    srcp = jnp.concatenate([edge_index[0], jnp.zeros((pad,), jnp.int32)])
    dstp = jnp.concatenate([edge_index[1], jnp.zeros((pad,), jnp.int32)])
    ewp = jnp.concatenate([edge_attr, jnp.zeros((pad,), jnp.float32)])

    dinv = _deg_dinv(dstp, ewp)
    xh0, xh1 = _glu_mm(x2, W1, b1, Wg)
    t = _gcn_agg(srcp, dstp, ewp, dinv, xh0, xh1, bg)
    y = _glu2(t, W2, b2)
    return y.reshape(1, 1, N, C)
